# baseline (speedup 1.0000x reference)
import functools

import jax
import jax.numpy as jnp
from jax.experimental import pallas as pl
from jax.experimental.pallas import tpu as pltpu

H = 128


def _silu(x):
    return x * jax.nn.sigmoid(x)


def _triplet_body(g_ref, af_ref, w1a_ref, w1b_ref, b1_ref, w2_ref, b2_ref, o_ref):
    h = (
        jnp.dot(g_ref[...], w1a_ref[...], preferred_element_type=jnp.float32)
        + jnp.dot(af_ref[...], w1b_ref[...], preferred_element_type=jnp.float32)
        + b1_ref[...]
    )
    h = _silu(h)
    o_ref[...] = (
        jnp.dot(h, w2_ref[...], preferred_element_type=jnp.float32) + b2_ref[...]
    )


def _triplet_mlp(gath, angle_feat, tW1, tb1, tW2, tb2, block=1600):
    T = gath.shape[0]
    grid = T // block
    w1a, w1b = tW1[:32], tW1[32:]
    return pl.pallas_call(
        _triplet_body,
        grid=(grid,),
        in_specs=[
            pl.BlockSpec((block, 32), lambda i: (i, 0)),
            pl.BlockSpec((block, 16), lambda i: (i, 0)),
            pl.BlockSpec((32, H), lambda i: (0, 0)),
            pl.BlockSpec((16, H), lambda i: (0, 0)),
            pl.BlockSpec((H,), lambda i: (0,)),
            pl.BlockSpec((H, H), lambda i: (0, 0)),
            pl.BlockSpec((H,), lambda i: (0,)),
        ],
        out_specs=pl.BlockSpec((block, H), lambda i: (i, 0)),
        out_shape=jax.ShapeDtypeStruct((T, H), jnp.float32),
    )(gath, angle_feat, w1a, w1b, tb1, tW2, tb2)


def _edge_body(g_ref, rbf_ref, agg_ref, w1r_ref, w1a_ref, b1_ref, w2_ref, b2_ref, o_ref):
    h = (
        g_ref[...]
        + jnp.dot(rbf_ref[...], w1r_ref[...], preferred_element_type=jnp.float32)
        + jnp.dot(agg_ref[...], w1a_ref[...], preferred_element_type=jnp.float32)
        + b1_ref[...]
    )
    h = _silu(h)
    o_ref[...] = (
        jnp.dot(h, w2_ref[...], preferred_element_type=jnp.float32) + b2_ref[...]
    )


def _edge_mlp(gath, rbf, agg, w1r, w1a, mb1, mW2, mb2, block=1600):
    E = gath.shape[0]
    grid = E // block
    return pl.pallas_call(
        _edge_body,
        grid=(grid,),
        in_specs=[
            pl.BlockSpec((block, H), lambda i: (i, 0)),
            pl.BlockSpec((block, 16), lambda i: (i, 0)),
            pl.BlockSpec((block, H), lambda i: (i, 0)),
            pl.BlockSpec((16, H), lambda i: (0, 0)),
            pl.BlockSpec((H, H), lambda i: (0, 0)),
            pl.BlockSpec((H,), lambda i: (0,)),
            pl.BlockSpec((H, 2 * H), lambda i: (0, 0)),
            pl.BlockSpec((2 * H,), lambda i: (0,)),
        ],
        out_specs=pl.BlockSpec((block, 2 * H), lambda i: (i, 0)),
        out_shape=jax.ShapeDtypeStruct((E, 2 * H), jnp.float32),
    )(gath, rbf, agg, w1r, w1a, mb1, mW2, mb2)


def _node_body(
    s_ref, aggs_ref, vT_ref, aggvT_ref,
    uw1s_ref, uw1a_ref, uw1n_ref, ub1_ref, uw2_ref, ub2_ref,
    gw1s_ref, gw1a_ref, gw1n_ref, gb1_ref, gw2_ref, gb2_ref,
    lng_ref, lnb_ref,
    so_ref, vo_ref,
):
    s = s_ref[...]
    aggs = aggs_ref[...]
    vx, vy, vz = vT_ref[0], vT_ref[1], vT_ref[2]
    vn = jnp.sqrt(vx * vx + vy * vy + vz * vz)

    def mlp(w1s, w1a, w1n, b1, w2, b2):
        h = (
            jnp.dot(s, w1s, preferred_element_type=jnp.float32)
            + jnp.dot(aggs, w1a, preferred_element_type=jnp.float32)
            + jnp.dot(vn, w1n, preferred_element_type=jnp.float32)
            + b1
        )
        h = _silu(h)
        return jnp.dot(h, w2, preferred_element_type=jnp.float32) + b2

    s_out = s + mlp(uw1s_ref[...], uw1a_ref[...], uw1n_ref[...], ub1_ref[...],
                    uw2_ref[...], ub2_ref[...])
    gate = jax.nn.sigmoid(
        mlp(gw1s_ref[...], gw1a_ref[...], gw1n_ref[...], gb1_ref[...],
            gw2_ref[...], gb2_ref[...])
    )
    mu = jnp.mean(s_out, axis=-1, keepdims=True)
    var = jnp.mean((s_out - mu) ** 2, axis=-1, keepdims=True)
    s_out = (s_out - mu) * jax.lax.rsqrt(var + 1e-5) * lng_ref[...] + lnb_ref[...]
    so_ref[...] = _silu(s_out)
    vo_ref[0] = vx + gate * aggvT_ref[0]
    vo_ref[1] = vy + gate * aggvT_ref[1]
    vo_ref[2] = vz + gate * aggvT_ref[2]


def _node_update(s, agg_s, vT, agg_vT, uW1, ub1, uW2, ub2, gW1, gb1, gW2, gb2,
                 ln_g, ln_b, block=1000):
    N = s.shape[0]
    grid = N // block
    wspec = pl.BlockSpec((H, H), lambda i: (0, 0))
    bspec = pl.BlockSpec((H,), lambda i: (0,))
    return pl.pallas_call(
        _node_body,
        grid=(grid,),
        in_specs=[
            pl.BlockSpec((block, H), lambda i: (i, 0)),
            pl.BlockSpec((block, H), lambda i: (i, 0)),
            pl.BlockSpec((3, block, H), lambda i: (0, i, 0)),
            pl.BlockSpec((3, block, H), lambda i: (0, i, 0)),
            wspec, wspec, wspec, bspec, wspec, bspec,
            wspec, wspec, wspec, bspec, wspec, bspec,
            bspec, bspec,
        ],
        out_specs=[
            pl.BlockSpec((block, H), lambda i: (i, 0)),
            pl.BlockSpec((3, block, H), lambda i: (0, i, 0)),
        ],
        out_shape=[
            jax.ShapeDtypeStruct((N, H), jnp.float32),
            jax.ShapeDtypeStruct((3, N, H), jnp.float32),
        ],
    )(
        s, agg_s, vT, agg_vT,
        uW1[:H], uW1[H:2 * H], uW1[2 * H:], ub1, uW2, ub2,
        gW1[:H], gW1[H:2 * H], gW1[2 * H:], gb1, gW2, gb2,
        ln_g, ln_b,
    )


def kernel(s, v, edge_index, edge_rbf, edge_dir, triplet_kj, triplet_ji,
           angle_feat, tW1, tb1, tW2, tb2, mW1, mb1, mW2, mb2,
           uW1, ub1, uW2, ub2, gW1, gb1, gW2, gb2, ln_g, ln_b):
    E = edge_rbf.shape[0]
    N = s.shape[0]

    gath = jnp.concatenate(
        [edge_rbf[triplet_kj], edge_rbf[triplet_ji]], axis=-1)
    t_msg = _triplet_mlp(gath, angle_feat, tW1, tb1, tW2, tb2)
    angle_agg = jnp.zeros((E, H), jnp.float32).at[triplet_ji].add(t_msg)

    src, dst = edge_index[0], edge_index[1]
    P1 = jnp.dot(s, mW1[:H])
    P2 = jnp.dot(s, mW1[H:2 * H])
    gath_e = P1[src] + P2[dst]
    msg = _edge_mlp(gath_e, edge_rbf, angle_agg, mW1[2 * H:2 * H + 16],
                    mW1[2 * H + 16:], mb1, mW2, mb2)
    m_s, m_v_coeff = msg[:, :H], msg[:, H:]

    agg_s = jnp.zeros((N, H), jnp.float32).at[dst].add(m_s)
    agg_vT = jnp.zeros((3, N, H), jnp.float32).at[:, dst].add(
        edge_dir.T[:, :, None] * m_v_coeff[None])

    vT = v.transpose(2, 0, 1)
    s_out, v_outT = _node_update(s, agg_s, vT, agg_vT, uW1, ub1, uW2, ub2,
                                 gW1, gb1, gW2, gb2, ln_g, ln_b)
    return s_out, v_outT.transpose(1, 2, 0)


# speedup vs baseline: 7.9335x; 7.9335x over previous
import functools

import jax
import jax.numpy as jnp
from jax import lax
from jax.experimental import pallas as pl
from jax.experimental.pallas import tpu as pltpu
from jax.experimental.pallas import tpu_sc as plsc

H = 128
_NC, _NS = 2, 16
_NW = _NC * _NS


def _sc_gather(table, idx, block):
    B = idx.shape[0]
    D = table.shape[1]
    b_per_w = B // _NW
    block = min(block, b_per_w)
    nchunk = b_per_w // block
    assert b_per_w % block == 0 and block % 8 == 0
    mesh = plsc.VectorSubcoreMesh(core_axis_name="c", subcore_axis_name="s")

    @functools.partial(
        pl.kernel,
        out_type=jax.ShapeDtypeStruct((B, D), jnp.float32),
        mesh=mesh,
        scratch_types=[
            pltpu.VMEM((block,), jnp.int32),
            pltpu.VMEM((block, D), jnp.float32),
            pltpu.SemaphoreType.DMA,
        ],
        compiler_params=pltpu.CompilerParams(use_tc_tiling_on_sc=False),
    )
    def k(table_hbm, idx_hbm, out_hbm, idx_v, rows_v, sem):
        wid = lax.axis_index("s") * _NC + lax.axis_index("c")
        base = wid * b_per_w

        def body(i, carry):
            off = base + i * block
            pltpu.sync_copy(idx_hbm.at[pl.ds(off, block)], idx_v)
            pltpu.async_copy(table_hbm.at[idx_v], rows_v, sem).wait()
            pltpu.sync_copy(rows_v, out_hbm.at[pl.ds(off, block)])
            return carry

        lax.fori_loop(0, nchunk, body, 0)

    return k(table, idx)


def _silu(x):
    return x * jax.nn.sigmoid(x)


def _triplet_body(gkj_ref, gji_ref, af_ref, w1a_ref, w1b_ref, w1c_ref,
                  b1_ref, w2_ref, b2_ref, o_ref):
    h = (
        jnp.dot(gkj_ref[...], w1a_ref[...], preferred_element_type=jnp.float32)
        + jnp.dot(gji_ref[...], w1b_ref[...], preferred_element_type=jnp.float32)
        + jnp.dot(af_ref[...], w1c_ref[...], preferred_element_type=jnp.float32)
        + b1_ref[...]
    )
    h = _silu(h)
    o_ref[...] = (
        jnp.dot(h, w2_ref[...], preferred_element_type=jnp.float32) + b2_ref[...]
    )


def _triplet_mlp(g_kj, g_ji, angle_feat, tW1, tb1, tW2, tb2, block=1600):
    T = g_kj.shape[0]
    grid = T // block
    nspec = pl.BlockSpec((block, 16), lambda i: (i, 0))
    wspec = pl.BlockSpec((16, H), lambda i: (0, 0))
    bspec = pl.BlockSpec((H,), lambda i: (0,))
    return pl.pallas_call(
        _triplet_body,
        grid=(grid,),
        in_specs=[
            nspec, nspec, nspec,
            wspec, wspec, wspec, bspec,
            pl.BlockSpec((H, H), lambda i: (0, 0)),
            bspec,
        ],
        out_specs=pl.BlockSpec((block, H), lambda i: (i, 0)),
        out_shape=jax.ShapeDtypeStruct((T, H), jnp.float32),
    )(g_kj, g_ji, angle_feat, tW1[:16], tW1[16:32], tW1[32:], tb1, tW2, tb2)


def _edge_body(gs_ref, gd_ref, rbf_ref, agg_ref, w1s_ref, w1d_ref, w1r_ref,
               w1a_ref, b1_ref, w2_ref, b2_ref, o_ref):
    h = (
        jnp.dot(gs_ref[...], w1s_ref[...], preferred_element_type=jnp.float32)
        + jnp.dot(gd_ref[...], w1d_ref[...], preferred_element_type=jnp.float32)
        + jnp.dot(rbf_ref[...], w1r_ref[...], preferred_element_type=jnp.float32)
        + jnp.dot(agg_ref[...], w1a_ref[...], preferred_element_type=jnp.float32)
        + b1_ref[...]
    )
    h = _silu(h)
    o_ref[...] = (
        jnp.dot(h, w2_ref[...], preferred_element_type=jnp.float32) + b2_ref[...]
    )


def _edge_mlp(gs, gd, rbf, agg, mW1, mb1, mW2, mb2, block=1600):
    E = gs.shape[0]
    grid = E // block
    hspec = pl.BlockSpec((block, H), lambda i: (i, 0))
    wspec = pl.BlockSpec((H, H), lambda i: (0, 0))
    return pl.pallas_call(
        _edge_body,
        grid=(grid,),
        in_specs=[
            hspec,
            hspec,
            pl.BlockSpec((block, 16), lambda i: (i, 0)),
            hspec,
            wspec, wspec,
            pl.BlockSpec((16, H), lambda i: (0, 0)),
            wspec,
            pl.BlockSpec((H,), lambda i: (0,)),
            pl.BlockSpec((H, 2 * H), lambda i: (0, 0)),
            pl.BlockSpec((2 * H,), lambda i: (0,)),
        ],
        out_specs=pl.BlockSpec((block, 2 * H), lambda i: (i, 0)),
        out_shape=jax.ShapeDtypeStruct((E, 2 * H), jnp.float32),
    )(gs, gd, rbf, agg, mW1[:H], mW1[H:2 * H], mW1[2 * H:2 * H + 16],
      mW1[2 * H + 16:], mb1, mW2, mb2)


def _node_body(
    s_ref, aggs_ref, vT_ref, aggvT_ref,
    uw1s_ref, uw1a_ref, uw1n_ref, ub1_ref, uw2_ref, ub2_ref,
    gw1s_ref, gw1a_ref, gw1n_ref, gb1_ref, gw2_ref, gb2_ref,
    lng_ref, lnb_ref,
    so_ref, vo_ref,
):
    s = s_ref[...]
    aggs = aggs_ref[...]
    vx, vy, vz = vT_ref[0], vT_ref[1], vT_ref[2]
    vn = jnp.sqrt(vx * vx + vy * vy + vz * vz)

    def mlp(w1s, w1a, w1n, b1, w2, b2):
        h = (
            jnp.dot(s, w1s, preferred_element_type=jnp.float32)
            + jnp.dot(aggs, w1a, preferred_element_type=jnp.float32)
            + jnp.dot(vn, w1n, preferred_element_type=jnp.float32)
            + b1
        )
        h = _silu(h)
        return jnp.dot(h, w2, preferred_element_type=jnp.float32) + b2

    s_out = s + mlp(uw1s_ref[...], uw1a_ref[...], uw1n_ref[...], ub1_ref[...],
                    uw2_ref[...], ub2_ref[...])
    gate = jax.nn.sigmoid(
        mlp(gw1s_ref[...], gw1a_ref[...], gw1n_ref[...], gb1_ref[...],
            gw2_ref[...], gb2_ref[...])
    )
    mu = jnp.mean(s_out, axis=-1, keepdims=True)
    var = jnp.mean((s_out - mu) ** 2, axis=-1, keepdims=True)
    s_out = (s_out - mu) * jax.lax.rsqrt(var + 1e-5) * lng_ref[...] + lnb_ref[...]
    so_ref[...] = _silu(s_out)
    vo_ref[0] = vx + gate * aggvT_ref[0]
    vo_ref[1] = vy + gate * aggvT_ref[1]
    vo_ref[2] = vz + gate * aggvT_ref[2]


def _node_update(s, agg_s, vT, agg_vT, uW1, ub1, uW2, ub2, gW1, gb1, gW2, gb2,
                 ln_g, ln_b, block=1000):
    N = s.shape[0]
    grid = N // block
    wspec = pl.BlockSpec((H, H), lambda i: (0, 0))
    bspec = pl.BlockSpec((H,), lambda i: (0,))
    return pl.pallas_call(
        _node_body,
        grid=(grid,),
        in_specs=[
            pl.BlockSpec((block, H), lambda i: (i, 0)),
            pl.BlockSpec((block, H), lambda i: (i, 0)),
            pl.BlockSpec((3, block, H), lambda i: (0, i, 0)),
            pl.BlockSpec((3, block, H), lambda i: (0, i, 0)),
            wspec, wspec, wspec, bspec, wspec, bspec,
            wspec, wspec, wspec, bspec, wspec, bspec,
            bspec, bspec,
        ],
        out_specs=[
            pl.BlockSpec((block, H), lambda i: (i, 0)),
            pl.BlockSpec((3, block, H), lambda i: (0, i, 0)),
        ],
        out_shape=[
            jax.ShapeDtypeStruct((N, H), jnp.float32),
            jax.ShapeDtypeStruct((3, N, H), jnp.float32),
        ],
    )(
        s, agg_s, vT, agg_vT,
        uW1[:H], uW1[H:2 * H], uW1[2 * H:], ub1, uW2, ub2,
        gW1[:H], gW1[H:2 * H], gW1[2 * H:], gb1, gW2, gb2,
        ln_g, ln_b,
    )


def kernel(s, v, edge_index, edge_rbf, edge_dir, triplet_kj, triplet_ji,
           angle_feat, tW1, tb1, tW2, tb2, mW1, mb1, mW2, mb2,
           uW1, ub1, uW2, ub2, gW1, gb1, gW2, gb2, ln_g, ln_b):
    E = edge_rbf.shape[0]
    N = s.shape[0]

    g_kj = _sc_gather(edge_rbf, triplet_kj, block=4000)
    g_ji = _sc_gather(edge_rbf, triplet_ji, block=4000)
    t_msg = _triplet_mlp(g_kj, g_ji, angle_feat, tW1, tb1, tW2, tb2)
    angle_agg = jnp.zeros((E, H), jnp.float32).at[triplet_ji].add(t_msg)

    src, dst = edge_index[0], edge_index[1]
    gs_src = _sc_gather(s, src, block=400)
    gs_dst = _sc_gather(s, dst, block=400)
    msg = _edge_mlp(gs_src, gs_dst, edge_rbf, angle_agg, mW1, mb1, mW2, mb2)
    m_s, m_v_coeff = msg[:, :H], msg[:, H:]

    agg_s = jnp.zeros((N, H), jnp.float32).at[dst].add(m_s)
    m_v = (m_v_coeff[:, :, None] * edge_dir[:, None, :]).reshape(E, H * 3)
    agg_v = jnp.zeros((N, H * 3), jnp.float32).at[dst].add(m_v)
    agg_vT = agg_v.reshape(N, H, 3).transpose(2, 0, 1)

    vT = v.transpose(2, 0, 1)
    s_out, v_outT = _node_update(s, agg_s, vT, agg_vT, uW1, ub1, uW2, ub2,
                                 gW1, gb1, gW2, gb2, ln_g, ln_b)
    return s_out, v_outT.transpose(1, 2, 0)


# speedup vs baseline: 9.7997x; 1.2352x over previous
import functools

import jax
import jax.numpy as jnp
from jax import lax
from jax.experimental import pallas as pl
from jax.experimental.pallas import tpu as pltpu
from jax.experimental.pallas import tpu_sc as plsc

H = 128
_NC, _NS = 2, 16
_NW = _NC * _NS


def _sc_gather2(table, idx_a, idx_b, block):
    B = idx_a.shape[0]
    D = table.shape[1]
    b_per_w = B // _NW
    block = min(block, b_per_w)
    nchunk = b_per_w // block
    assert b_per_w % block == 0 and block % 8 == 0
    mesh = plsc.VectorSubcoreMesh(core_axis_name="c", subcore_axis_name="s")
    out = jax.ShapeDtypeStruct((B, D), jnp.float32)

    @functools.partial(
        pl.kernel,
        out_type=(out, out),
        mesh=mesh,
        scratch_types=[
            pltpu.VMEM((block,), jnp.int32),
            pltpu.VMEM((block,), jnp.int32),
            pltpu.VMEM((block, D), jnp.float32),
            pltpu.VMEM((block, D), jnp.float32),
            pltpu.SemaphoreType.DMA,
            pltpu.SemaphoreType.DMA,
        ],
        compiler_params=pltpu.CompilerParams(use_tc_tiling_on_sc=False),
    )
    def k(table_hbm, ia_hbm, ib_hbm, oa_hbm, ob_hbm,
          ia_v, ib_v, ra_v, rb_v, sa, sb):
        wid = lax.axis_index("s") * _NC + lax.axis_index("c")
        base = wid * b_per_w

        def body(i, carry):
            off = base + i * block
            pltpu.sync_copy(ia_hbm.at[pl.ds(off, block)], ia_v)
            cpa = pltpu.async_copy(table_hbm.at[ia_v], ra_v, sa)
            pltpu.sync_copy(ib_hbm.at[pl.ds(off, block)], ib_v)
            cpb = pltpu.async_copy(table_hbm.at[ib_v], rb_v, sb)
            cpa.wait()
            pltpu.sync_copy(ra_v, oa_hbm.at[pl.ds(off, block)])
            cpb.wait()
            pltpu.sync_copy(rb_v, ob_hbm.at[pl.ds(off, block)])
            return carry

        lax.fori_loop(0, nchunk, body, 0)

    return k(table, idx_a, idx_b)


def _silu(x):
    return x * jax.nn.sigmoid(x)


def _triplet_body(gkj_ref, gji_ref, af_ref, w1a_ref, w1b_ref, w1c_ref,
                  b1_ref, w2_ref, b2_ref, o_ref):
    h = (
        jnp.dot(gkj_ref[...], w1a_ref[...], preferred_element_type=jnp.float32)
        + jnp.dot(gji_ref[...], w1b_ref[...], preferred_element_type=jnp.float32)
        + jnp.dot(af_ref[...], w1c_ref[...], preferred_element_type=jnp.float32)
        + b1_ref[...]
    )
    h = _silu(h)
    o_ref[...] = (
        jnp.dot(h, w2_ref[...], preferred_element_type=jnp.float32) + b2_ref[...]
    )


def _triplet_mlp(g_kj, g_ji, angle_feat, tW1, tb1, tW2, tb2, block=1600):
    T = g_kj.shape[0]
    grid = T // block
    nspec = pl.BlockSpec((block, 16), lambda i: (i, 0))
    wspec = pl.BlockSpec((16, H), lambda i: (0, 0))
    bspec = pl.BlockSpec((H,), lambda i: (0,))
    return pl.pallas_call(
        _triplet_body,
        grid=(grid,),
        in_specs=[
            nspec, nspec, nspec,
            wspec, wspec, wspec, bspec,
            pl.BlockSpec((H, H), lambda i: (0, 0)),
            bspec,
        ],
        out_specs=pl.BlockSpec((block, H), lambda i: (i, 0)),
        out_shape=jax.ShapeDtypeStruct((T, H), jnp.float32),
    )(g_kj, g_ji, angle_feat, tW1[:16], tW1[16:32], tW1[32:], tb1, tW2, tb2)


def _edge_body(gs_ref, gd_ref, rbf_ref, agg_ref, dir_ref, w1s_ref, w1d_ref,
               w1r_ref, w1a_ref, b1_ref, w2_ref, b2_ref, o_ref):
    h = (
        jnp.dot(gs_ref[...], w1s_ref[...], preferred_element_type=jnp.float32)
        + jnp.dot(gd_ref[...], w1d_ref[...], preferred_element_type=jnp.float32)
        + jnp.dot(rbf_ref[...], w1r_ref[...], preferred_element_type=jnp.float32)
        + jnp.dot(agg_ref[...], w1a_ref[...], preferred_element_type=jnp.float32)
        + b1_ref[...]
    )
    h = _silu(h)
    msg = jnp.dot(h, w2_ref[...], preferred_element_type=jnp.float32) + b2_ref[...]
    m_s, coeff = msg[:, :H], msg[:, H:]
    d = dir_ref[...]
    o_ref[...] = jnp.concatenate(
        [m_s, coeff * d[:, 0:1], coeff * d[:, 1:2], coeff * d[:, 2:3]], axis=-1)


def _edge_mlp(gs, gd, rbf, agg, edge_dir, mW1, mb1, mW2, mb2, block=1600):
    E = gs.shape[0]
    grid = E // block
    hspec = pl.BlockSpec((block, H), lambda i: (i, 0))
    wspec = pl.BlockSpec((H, H), lambda i: (0, 0))
    return pl.pallas_call(
        _edge_body,
        grid=(grid,),
        in_specs=[
            hspec,
            hspec,
            pl.BlockSpec((block, 16), lambda i: (i, 0)),
            hspec,
            pl.BlockSpec((block, 3), lambda i: (i, 0)),
            wspec, wspec,
            pl.BlockSpec((16, H), lambda i: (0, 0)),
            wspec,
            pl.BlockSpec((H,), lambda i: (0,)),
            pl.BlockSpec((H, 2 * H), lambda i: (0, 0)),
            pl.BlockSpec((2 * H,), lambda i: (0,)),
        ],
        out_specs=pl.BlockSpec((block, 4 * H), lambda i: (i, 0)),
        out_shape=jax.ShapeDtypeStruct((E, 4 * H), jnp.float32),
    )(gs, gd, rbf, agg, edge_dir, mW1[:H], mW1[H:2 * H], mW1[2 * H:2 * H + 16],
      mW1[2 * H + 16:], mb1, mW2, mb2)


def _node_body(
    s_ref, agg_ref, vT_ref,
    uw1s_ref, uw1a_ref, uw1n_ref, ub1_ref, uw2_ref, ub2_ref,
    gw1s_ref, gw1a_ref, gw1n_ref, gb1_ref, gw2_ref, gb2_ref,
    lng_ref, lnb_ref,
    so_ref, vo_ref,
):
    s = s_ref[...]
    aggs = agg_ref[:, :H]
    vx, vy, vz = vT_ref[0], vT_ref[1], vT_ref[2]
    vn = jnp.sqrt(vx * vx + vy * vy + vz * vz)

    def mlp(w1s, w1a, w1n, b1, w2, b2):
        h = (
            jnp.dot(s, w1s, preferred_element_type=jnp.float32)
            + jnp.dot(aggs, w1a, preferred_element_type=jnp.float32)
            + jnp.dot(vn, w1n, preferred_element_type=jnp.float32)
            + b1
        )
        h = _silu(h)
        return jnp.dot(h, w2, preferred_element_type=jnp.float32) + b2

    s_out = s + mlp(uw1s_ref[...], uw1a_ref[...], uw1n_ref[...], ub1_ref[...],
                    uw2_ref[...], ub2_ref[...])
    gate = jax.nn.sigmoid(
        mlp(gw1s_ref[...], gw1a_ref[...], gw1n_ref[...], gb1_ref[...],
            gw2_ref[...], gb2_ref[...])
    )
    mu = jnp.mean(s_out, axis=-1, keepdims=True)
    var = jnp.mean((s_out - mu) ** 2, axis=-1, keepdims=True)
    s_out = (s_out - mu) * jax.lax.rsqrt(var + 1e-5) * lng_ref[...] + lnb_ref[...]
    so_ref[...] = _silu(s_out)
    vo_ref[0] = vx + gate * agg_ref[:, H:2 * H]
    vo_ref[1] = vy + gate * agg_ref[:, 2 * H:3 * H]
    vo_ref[2] = vz + gate * agg_ref[:, 3 * H:]


def _node_update(s, agg, vT, uW1, ub1, uW2, ub2, gW1, gb1, gW2, gb2,
                 ln_g, ln_b, block=1000):
    N = s.shape[0]
    grid = N // block
    wspec = pl.BlockSpec((H, H), lambda i: (0, 0))
    bspec = pl.BlockSpec((H,), lambda i: (0,))
    return pl.pallas_call(
        _node_body,
        grid=(grid,),
        in_specs=[
            pl.BlockSpec((block, H), lambda i: (i, 0)),
            pl.BlockSpec((block, 4 * H), lambda i: (i, 0)),
            pl.BlockSpec((3, block, H), lambda i: (0, i, 0)),
            wspec, wspec, wspec, bspec, wspec, bspec,
            wspec, wspec, wspec, bspec, wspec, bspec,
            bspec, bspec,
        ],
        out_specs=[
            pl.BlockSpec((block, H), lambda i: (i, 0)),
            pl.BlockSpec((3, block, H), lambda i: (0, i, 0)),
        ],
        out_shape=[
            jax.ShapeDtypeStruct((N, H), jnp.float32),
            jax.ShapeDtypeStruct((3, N, H), jnp.float32),
        ],
    )(
        s, agg, vT,
        uW1[:H], uW1[H:2 * H], uW1[2 * H:], ub1, uW2, ub2,
        gW1[:H], gW1[H:2 * H], gW1[2 * H:], gb1, gW2, gb2,
        ln_g, ln_b,
    )


def kernel(s, v, edge_index, edge_rbf, edge_dir, triplet_kj, triplet_ji,
           angle_feat, tW1, tb1, tW2, tb2, mW1, mb1, mW2, mb2,
           uW1, ub1, uW2, ub2, gW1, gb1, gW2, gb2, ln_g, ln_b):
    E = edge_rbf.shape[0]
    N = s.shape[0]

    g_kj, g_ji = _sc_gather2(edge_rbf, triplet_kj, triplet_ji, block=2000)
    t_msg = _triplet_mlp(g_kj, g_ji, angle_feat, tW1, tb1, tW2, tb2)
    angle_agg = jnp.zeros((E, H), jnp.float32).at[triplet_ji].add(t_msg)

    src, dst = edge_index[0], edge_index[1]
    gs_src, gs_dst = _sc_gather2(s, src, dst, block=400)
    pay = _edge_mlp(gs_src, gs_dst, edge_rbf, angle_agg, edge_dir,
                    mW1, mb1, mW2, mb2)

    agg = jnp.zeros((N, 4 * H), jnp.float32).at[dst].add(pay)

    vT = v.transpose(2, 0, 1)
    s_out, v_outT = _node_update(s, agg, vT, uW1, ub1, uW2, ub2,
                                 gW1, gb1, gW2, gb2, ln_g, ln_b)
    return s_out, v_outT.transpose(1, 2, 0)


# speedup vs baseline: 11.3688x; 1.1601x over previous
import functools

import jax
import jax.numpy as jnp
from jax import lax
from jax.experimental import pallas as pl
from jax.experimental.pallas import tpu as pltpu
from jax.experimental.pallas import tpu_sc as plsc

H = 128
_NC, _NS = 2, 16
_NW = _NC * _NS


def _sc_gather2(table, idx_a, idx_b, block):
    B = idx_a.shape[0]
    D = table.shape[1]
    b_per_w = B // _NW
    block = min(block, b_per_w)
    nchunk = b_per_w // block
    assert b_per_w % block == 0 and block % 8 == 0
    mesh = plsc.VectorSubcoreMesh(core_axis_name="c", subcore_axis_name="s")
    out = jax.ShapeDtypeStruct((B, D), jnp.float32)

    @functools.partial(
        pl.kernel,
        out_type=(out, out),
        mesh=mesh,
        scratch_types=[
            pltpu.VMEM((block,), jnp.int32),
            pltpu.VMEM((block,), jnp.int32),
            pltpu.VMEM((block, D), jnp.float32),
            pltpu.VMEM((block, D), jnp.float32),
            pltpu.SemaphoreType.DMA,
            pltpu.SemaphoreType.DMA,
        ],
        compiler_params=pltpu.CompilerParams(use_tc_tiling_on_sc=False),
    )
    def k(table_hbm, ia_hbm, ib_hbm, oa_hbm, ob_hbm,
          ia_v, ib_v, ra_v, rb_v, sa, sb):
        wid = lax.axis_index("s") * _NC + lax.axis_index("c")
        base = wid * b_per_w

        def body(i, carry):
            off = base + i * block
            pltpu.sync_copy(ia_hbm.at[pl.ds(off, block)], ia_v)
            cpa = pltpu.async_copy(table_hbm.at[ia_v], ra_v, sa)
            pltpu.sync_copy(ib_hbm.at[pl.ds(off, block)], ib_v)
            cpb = pltpu.async_copy(table_hbm.at[ib_v], rb_v, sb)
            cpa.wait()
            pltpu.sync_copy(ra_v, oa_hbm.at[pl.ds(off, block)])
            cpb.wait()
            pltpu.sync_copy(rb_v, ob_hbm.at[pl.ds(off, block)])
            return carry

        lax.fori_loop(0, nchunk, body, 0)

    return k(table, idx_a, idx_b)


def _silu(x):
    return x * jax.nn.sigmoid(x)


def _sc_node_scatter(pay, dst, N):
    E = pay.shape[0]
    C = 128
    nchunks = E // C
    base_k, extra = nchunks // _NS, nchunks % _NS
    rows_w = N // _NS
    zeros = jnp.zeros((N, H), jnp.float32)
    mesh = plsc.VectorSubcoreMesh(core_axis_name="c", subcore_axis_name="s")

    @functools.partial(
        pl.kernel,
        out_type=jax.ShapeDtypeStruct((N, 4 * H), jnp.float32),
        mesh=mesh,
        scratch_types=[
            pltpu.VMEM_SHARED((N, H), jnp.float32),
            pltpu.VMEM((C,), jnp.int32),
            pltpu.VMEM((C, H), jnp.float32),
        ],
        compiler_params=pltpu.CompilerParams(use_tc_tiling_on_sc=False),
    )
    def k(pay_hbm, dst_hbm, zero_hbm, out_hbm, acc, idx_v, pay_v):
        c = lax.axis_index("c")
        sid = lax.axis_index("s")
        nk = base_k + jnp.where(sid < extra, 1, 0)
        r0 = sid * rows_w

        for p in range(2):
            g = 2 * c + p
            col = g * H
            pltpu.sync_copy(zero_hbm.at[pl.ds(r0, rows_w)],
                            acc.at[pl.ds(r0, rows_w)])
            plsc.subcore_barrier()

            def body(j, carry):
                k_ = sid + j * _NS
                off = k_ * C
                pltpu.sync_copy(dst_hbm.at[pl.ds(off, C)], idx_v)
                pltpu.sync_copy(pay_hbm.at[pl.ds(off, C), pl.ds(col, H)],
                                pay_v)
                pltpu.sync_copy(pay_v, acc.at[idx_v], add=True)
                return carry

            lax.fori_loop(0, nk, body, 0)
            plsc.subcore_barrier()
            pltpu.sync_copy(acc.at[pl.ds(r0, rows_w)],
                            out_hbm.at[pl.ds(r0, rows_w), pl.ds(col, H)])
            plsc.subcore_barrier()

    return k(pay, dst, zeros)


def _triplet_body(gkj_ref, gji_ref, af_ref, w1a_ref, w1b_ref, w1c_ref,
                  b1_ref, w2_ref, b2_ref, o_ref):
    h = (
        jnp.dot(gkj_ref[...], w1a_ref[...], preferred_element_type=jnp.float32)
        + jnp.dot(gji_ref[...], w1b_ref[...], preferred_element_type=jnp.float32)
        + jnp.dot(af_ref[...], w1c_ref[...], preferred_element_type=jnp.float32)
        + b1_ref[...]
    )
    h = _silu(h)
    o_ref[...] = (
        jnp.dot(h, w2_ref[...], preferred_element_type=jnp.float32) + b2_ref[...]
    )


def _triplet_mlp(g_kj, g_ji, angle_feat, tW1, tb1, tW2, tb2, block=1600):
    T = g_kj.shape[0]
    grid = T // block
    nspec = pl.BlockSpec((block, 16), lambda i: (i, 0))
    wspec = pl.BlockSpec((16, H), lambda i: (0, 0))
    bspec = pl.BlockSpec((H,), lambda i: (0,))
    return pl.pallas_call(
        _triplet_body,
        grid=(grid,),
        in_specs=[
            nspec, nspec, nspec,
            wspec, wspec, wspec, bspec,
            pl.BlockSpec((H, H), lambda i: (0, 0)),
            bspec,
        ],
        out_specs=pl.BlockSpec((block, H), lambda i: (i, 0)),
        out_shape=jax.ShapeDtypeStruct((T, H), jnp.float32),
    )(g_kj, g_ji, angle_feat, tW1[:16], tW1[16:32], tW1[32:], tb1, tW2, tb2)


def _edge_body(gs_ref, gd_ref, rbf_ref, agg_ref, dir_ref, w1s_ref, w1d_ref,
               w1r_ref, w1a_ref, b1_ref, w2_ref, b2_ref, o_ref):
    h = (
        jnp.dot(gs_ref[...], w1s_ref[...], preferred_element_type=jnp.float32)
        + jnp.dot(gd_ref[...], w1d_ref[...], preferred_element_type=jnp.float32)
        + jnp.dot(rbf_ref[...], w1r_ref[...], preferred_element_type=jnp.float32)
        + jnp.dot(agg_ref[...], w1a_ref[...], preferred_element_type=jnp.float32)
        + b1_ref[...]
    )
    h = _silu(h)
    msg = jnp.dot(h, w2_ref[...], preferred_element_type=jnp.float32) + b2_ref[...]
    m_s, coeff = msg[:, :H], msg[:, H:]
    d = dir_ref[...]
    o_ref[...] = jnp.concatenate(
        [m_s, coeff * d[:, 0:1], coeff * d[:, 1:2], coeff * d[:, 2:3]], axis=-1)


def _edge_mlp(gs, gd, rbf, agg, edge_dir, mW1, mb1, mW2, mb2, block=1600):
    E = gs.shape[0]
    grid = E // block
    hspec = pl.BlockSpec((block, H), lambda i: (i, 0))
    wspec = pl.BlockSpec((H, H), lambda i: (0, 0))
    return pl.pallas_call(
        _edge_body,
        grid=(grid,),
        in_specs=[
            hspec,
            hspec,
            pl.BlockSpec((block, 16), lambda i: (i, 0)),
            hspec,
            pl.BlockSpec((block, 3), lambda i: (i, 0)),
            wspec, wspec,
            pl.BlockSpec((16, H), lambda i: (0, 0)),
            wspec,
            pl.BlockSpec((H,), lambda i: (0,)),
            pl.BlockSpec((H, 2 * H), lambda i: (0, 0)),
            pl.BlockSpec((2 * H,), lambda i: (0,)),
        ],
        out_specs=pl.BlockSpec((block, 4 * H), lambda i: (i, 0)),
        out_shape=jax.ShapeDtypeStruct((E, 4 * H), jnp.float32),
    )(gs, gd, rbf, agg, edge_dir, mW1[:H], mW1[H:2 * H], mW1[2 * H:2 * H + 16],
      mW1[2 * H + 16:], mb1, mW2, mb2)


def _node_body(
    s_ref, agg_ref, vT_ref,
    uw1s_ref, uw1a_ref, uw1n_ref, ub1_ref, uw2_ref, ub2_ref,
    gw1s_ref, gw1a_ref, gw1n_ref, gb1_ref, gw2_ref, gb2_ref,
    lng_ref, lnb_ref,
    so_ref, vo_ref,
):
    s = s_ref[...]
    aggs = agg_ref[:, :H]
    vx, vy, vz = vT_ref[0], vT_ref[1], vT_ref[2]
    vn = jnp.sqrt(vx * vx + vy * vy + vz * vz)

    def mlp(w1s, w1a, w1n, b1, w2, b2):
        h = (
            jnp.dot(s, w1s, preferred_element_type=jnp.float32)
            + jnp.dot(aggs, w1a, preferred_element_type=jnp.float32)
            + jnp.dot(vn, w1n, preferred_element_type=jnp.float32)
            + b1
        )
        h = _silu(h)
        return jnp.dot(h, w2, preferred_element_type=jnp.float32) + b2

    s_out = s + mlp(uw1s_ref[...], uw1a_ref[...], uw1n_ref[...], ub1_ref[...],
                    uw2_ref[...], ub2_ref[...])
    gate = jax.nn.sigmoid(
        mlp(gw1s_ref[...], gw1a_ref[...], gw1n_ref[...], gb1_ref[...],
            gw2_ref[...], gb2_ref[...])
    )
    mu = jnp.mean(s_out, axis=-1, keepdims=True)
    var = jnp.mean((s_out - mu) ** 2, axis=-1, keepdims=True)
    s_out = (s_out - mu) * jax.lax.rsqrt(var + 1e-5) * lng_ref[...] + lnb_ref[...]
    so_ref[...] = _silu(s_out)
    vo_ref[0] = vx + gate * agg_ref[:, H:2 * H]
    vo_ref[1] = vy + gate * agg_ref[:, 2 * H:3 * H]
    vo_ref[2] = vz + gate * agg_ref[:, 3 * H:]


def _node_update(s, agg, vT, uW1, ub1, uW2, ub2, gW1, gb1, gW2, gb2,
                 ln_g, ln_b, block=1000):
    N = s.shape[0]
    grid = N // block
    wspec = pl.BlockSpec((H, H), lambda i: (0, 0))
    bspec = pl.BlockSpec((H,), lambda i: (0,))
    return pl.pallas_call(
        _node_body,
        grid=(grid,),
        in_specs=[
            pl.BlockSpec((block, H), lambda i: (i, 0)),
            pl.BlockSpec((block, 4 * H), lambda i: (i, 0)),
            pl.BlockSpec((3, block, H), lambda i: (0, i, 0)),
            wspec, wspec, wspec, bspec, wspec, bspec,
            wspec, wspec, wspec, bspec, wspec, bspec,
            bspec, bspec,
        ],
        out_specs=[
            pl.BlockSpec((block, H), lambda i: (i, 0)),
            pl.BlockSpec((3, block, H), lambda i: (0, i, 0)),
        ],
        out_shape=[
            jax.ShapeDtypeStruct((N, H), jnp.float32),
            jax.ShapeDtypeStruct((3, N, H), jnp.float32),
        ],
    )(
        s, agg, vT,
        uW1[:H], uW1[H:2 * H], uW1[2 * H:], ub1, uW2, ub2,
        gW1[:H], gW1[H:2 * H], gW1[2 * H:], gb1, gW2, gb2,
        ln_g, ln_b,
    )


def kernel(s, v, edge_index, edge_rbf, edge_dir, triplet_kj, triplet_ji,
           angle_feat, tW1, tb1, tW2, tb2, mW1, mb1, mW2, mb2,
           uW1, ub1, uW2, ub2, gW1, gb1, gW2, gb2, ln_g, ln_b):
    E = edge_rbf.shape[0]
    N = s.shape[0]

    g_kj, g_ji = _sc_gather2(edge_rbf, triplet_kj, triplet_ji, block=2000)
    t_msg = _triplet_mlp(g_kj, g_ji, angle_feat, tW1, tb1, tW2, tb2)
    angle_agg = jnp.zeros((E, H), jnp.float32).at[triplet_ji].add(t_msg)

    src, dst = edge_index[0], edge_index[1]
    gs_src, gs_dst = _sc_gather2(s, src, dst, block=400)
    pay = _edge_mlp(gs_src, gs_dst, edge_rbf, angle_agg, edge_dir,
                    mW1, mb1, mW2, mb2)

    agg = _sc_node_scatter(pay, dst, N)

    vT = v.transpose(2, 0, 1)
    s_out, v_outT = _node_update(s, agg, vT, uW1, ub1, uW2, ub2,
                                 gW1, gb1, gW2, gb2, ln_g, ln_b)
    return s_out, v_outT.transpose(1, 2, 0)
